# baseline (speedup 1.0000x reference)
import functools

import jax
import jax.numpy as jnp
from jax import lax
from jax.experimental import pallas as pl
from jax.experimental.pallas import tpu as pltpu
from jax.experimental.pallas import tpu_sc as plsc

N = 10000
D = 128
E = 320000
CNT_LANES = 16
W_AUG = D + CNT_LANES

NC = 2
NS = 16
NW = NC * NS
CHUNK = 64
SBC = 8
K = 160
SB = K // SBC
NSB = SB + 1
NO = SB // 2
E_PAD = NW * CHUNK * K
N_ACC = 10112
ROWS_PER_TILE = N_ACC // NS


def _sc_agg_body(table, idx_hbm, out, rows0, rows1, rows2, rows3, ib0, ib1,
                 acc, is0, is1, gs0, gs1, gs2, gs3, ss0, ss1, ss2, ss3):
    c = lax.axis_index("c")
    s = lax.axis_index("s")
    wid = s * NC + c
    rows = [rows0, rows1, rows2, rows3]
    ib = [ib0, ib1]
    gsem = [gs0, gs1, gs2, gs3]
    ssem = [ss0, ss1, ss2, ss3]

    cp0 = pltpu.async_copy(idx_hbm.at[wid, 0], ib0, is0)
    cp1 = pltpu.async_copy(idx_hbm.at[wid, 1], ib1, is1)

    @pl.loop(0, CHUNK)
    def _(i):
        @pl.loop(0, W_AUG, step=16)
        def _(j):
            rows0[i, pl.ds(j, 16)] = jnp.zeros((16,), jnp.float32)

    base = s * ROWS_PER_TILE
    nz = ROWS_PER_TILE // CHUNK
    rem = ROWS_PER_TILE - nz * CHUNK

    @pl.loop(0, nz)
    def _(q):
        pltpu.sync_copy(rows0, acc.at[pl.ds(base + q * CHUNK, CHUNK)])
    if rem:
        pltpu.sync_copy(rows0.at[pl.ds(0, rem)],
                        acc.at[pl.ds(base + nz * CHUNK, rem)])

    cp0.wait()
    cp1.wait()
    plsc.subcore_barrier()

    def gather(j_chunk_ib, row, slot):
        pltpu.async_copy(table.at[j_chunk_ib.at[row, 0]], rows[slot],
                         gsem[slot])

    def step(o, p, c, first):
        b = (p * SBC + c) % 4
        b2 = (b + 2) % 4
        cur = ib[p]
        pltpu.make_async_copy(table.at[cur.at[c, 0]], rows[b], gsem[b]).wait()
        pltpu.async_copy(rows[b], acc.at[cur.at[c, 1]], ssem[b], add=True)
        if not (first and p == 0 and c < 2):
            pltpu.make_async_copy(rows[b2], acc.at[cur.at[c, 1]],
                                  ssem[b2]).wait()
        if c == 2:
            if p == 0:
                if not first:
                    pltpu.async_copy(idx_hbm.at[wid, 2 * o + 1], ib1, is1)
            else:
                pltpu.async_copy(idx_hbm.at[wid, 2 * o + 2], ib0, is0)
        if c == 6:
            if p == 0:
                if not first:
                    pltpu.make_async_copy(idx_hbm.at[wid, 2 * o + 1], ib1,
                                          is1).wait()
            else:
                pltpu.make_async_copy(idx_hbm.at[wid, 2 * o + 2], ib0,
                                      is0).wait()
        if c < 6:
            gather(ib[p], c + 2, b2)
        else:
            gather(ib[1 - p], c - 6, b2)

    gather(ib0, 0, 0)
    gather(ib0, 1, 1)

    for p in range(2):
        for cc in range(SBC):
            step(0, p, cc, first=True)

    @pl.loop(1, NO)
    def _(o):
        for p in range(2):
            for cc in range(SBC):
                step(o, p, cc, first=False)

    pltpu.make_async_copy(table.at[ib0.at[0, 0]], rows0, gsem[0]).wait()
    pltpu.make_async_copy(table.at[ib0.at[1, 0]], rows1, gsem[1]).wait()
    pltpu.make_async_copy(rows2, acc.at[ib1.at[6, 1]], ssem[2]).wait()
    pltpu.make_async_copy(rows3, acc.at[ib1.at[7, 1]], ssem[3]).wait()

    plsc.subcore_barrier()

    pltpu.sync_copy(acc.at[pl.ds(base, ROWS_PER_TILE)],
                    out.at[c, pl.ds(base, ROWS_PER_TILE)])


@jax.jit
def _sc_agg(table, idx_hbm):
    mesh = plsc.VectorSubcoreMesh(core_axis_name="c", subcore_axis_name="s")
    kfn = pl.kernel(
        _sc_agg_body,
        out_type=jax.ShapeDtypeStruct((NC, N_ACC, W_AUG), jnp.float32),
        mesh=mesh,
        scratch_types=(
            [pltpu.VMEM((CHUNK, W_AUG), jnp.float32) for _ in range(4)]
            + [pltpu.VMEM((SBC, 2, CHUNK), jnp.int32) for _ in range(2)]
            + [pltpu.VMEM_SHARED((N_ACC, W_AUG), jnp.float32)]
            + [pltpu.SemaphoreType.DMA for _ in range(10)]
        ),
        compiler_params=pltpu.CompilerParams(use_tc_tiling_on_sc=False),
    )
    return kfn(table, idx_hbm)


def _tc_pre_body(x_ref, wl_ref, wr_ref, b_ref, yaug_ref, r_ref):
    xv = x_ref[...]
    yaug_ref[:, :D] = jnp.dot(xv, wl_ref[...], preferred_element_type=jnp.float32)
    yaug_ref[:, D:] = jnp.ones((N, CNT_LANES), jnp.float32)
    r_ref[...] = jnp.dot(xv, wr_ref[...], preferred_element_type=jnp.float32) + b_ref[...]


@jax.jit
def _tc_pre(x, wl_t, wr_t, b):
    return pl.pallas_call(
        _tc_pre_body,
        out_shape=(jax.ShapeDtypeStruct((N, W_AUG), jnp.float32),
                   jax.ShapeDtypeStruct((N, D), jnp.float32)),
    )(x, wl_t, wr_t, b)


def _mean_from_partials(p_ref):
    ssum = p_ref[0, :N, :D] + p_ref[1, :N, :D]
    cnt16 = p_ref[0, :N, D:] + p_ref[1, :N, D:]
    cnt = jnp.max(cnt16, axis=1, keepdims=True)
    return ssum / jnp.maximum(cnt, 1.0)


def _tc_mid_body(p_ref, r1_ref, wl_ref, wr_ref, b_ref, yaug_ref, r2_ref):
    h = jnp.maximum(_mean_from_partials(p_ref) + r1_ref[...], 0.0)
    yaug_ref[:, :D] = jnp.dot(h, wl_ref[...], preferred_element_type=jnp.float32)
    yaug_ref[:, D:] = jnp.ones((N, CNT_LANES), jnp.float32)
    r2_ref[...] = jnp.dot(h, wr_ref[...], preferred_element_type=jnp.float32) + b_ref[...]


@jax.jit
def _tc_mid(p1, r1, wl_t, wr_t, b):
    return pl.pallas_call(
        _tc_mid_body,
        out_shape=(jax.ShapeDtypeStruct((N, W_AUG), jnp.float32),
                   jax.ShapeDtypeStruct((N, D), jnp.float32)),
    )(p1, r1, wl_t, wr_t, b)


def _tc_post_body(p_ref, r2_ref, out_ref):
    out_ref[...] = _mean_from_partials(p_ref) + r2_ref[...]


@jax.jit
def _tc_post(p2, r2):
    return pl.pallas_call(
        _tc_post_body,
        out_shape=jax.ShapeDtypeStruct((N, D), jnp.float32),
    )(p2, r2)


def kernel(x, edgeIndex, W1_l, b1_l, W1_r, W2_l, b2_l, W2_r):
    src = edgeIndex[0]
    dst = edgeIndex[1]
    pad = E_PAD - E
    ntail = NW * SBC * CHUNK
    pad_src = (jnp.arange(pad, dtype=src.dtype) * 37) % N
    tail_src = (jnp.arange(ntail, dtype=src.dtype) * 53) % N
    srcp = jnp.concatenate([src, pad_src]).reshape(NW, K, CHUNK)
    srcp = jnp.concatenate([srcp, tail_src.reshape(NW, SBC, CHUNK)], axis=1)
    pad_dst = N + (jnp.arange(pad, dtype=dst.dtype) % (N_ACC - N))
    tail_dst = N + (jnp.arange(ntail, dtype=dst.dtype) % (N_ACC - N))
    dstp = jnp.concatenate([dst, pad_dst]).reshape(NW, K, CHUNK)
    dstp = jnp.concatenate([dstp, tail_dst.reshape(NW, SBC, CHUNK)], axis=1)
    idx = jnp.stack([srcp, dstp], axis=2)
    idx = idx.reshape(NW, NSB, SBC, 2, CHUNK)

    yaug1, r1 = _tc_pre(x, W1_l.T, W1_r.T, b1_l[None, :])
    p1 = _sc_agg(yaug1, idx)
    yaug2, r2 = _tc_mid(p1, r1, W2_l.T, W2_r.T, b2_l[None, :])
    p2 = _sc_agg(yaug2, idx)
    return _tc_post(p2, r2)


# speedup vs baseline: 1.4204x; 1.4204x over previous
import jax
import jax.numpy as jnp
from jax import lax
from jax.experimental import pallas as pl
from jax.experimental.pallas import tpu as pltpu
from jax.experimental.pallas import tpu_sc as plsc

N = 10000
D = 128
E = 320000

NC = 2
NS = 16
NW = NC * NS
CHUNK = 128
SBC = 8
K = 80
SB = K // SBC
NSB = SB + 1
NO = SB // 2
E_PAD = NW * CHUNK * K
N_ACC = 10112
ROWS_PER_TILE = N_ACC // NS
CNT_W = 16


def _sc_agg_body(table, src_hbm, dst_hbm, out, rows0, rows1, sib0, sib1,
                 dib0, dib1, acc, is0, is1, gs0, gs1):
    c = lax.axis_index("c")
    s = lax.axis_index("s")
    wid = s * NC + c
    rows = [rows0, rows1]
    sib = [sib0, sib1]
    dib = [dib0, dib1]
    isem = [is0, is1]
    gsem = [gs0, gs1]

    cps = [pltpu.async_copy(src_hbm.at[wid, 0], sib0, is0),
           pltpu.async_copy(dst_hbm.at[wid, 0], dib0, is0),
           pltpu.async_copy(src_hbm.at[wid, 1], sib1, is1),
           pltpu.async_copy(dst_hbm.at[wid, 1], dib1, is1)]

    @pl.loop(0, CHUNK)
    def _(i):
        @pl.loop(0, D, step=16)
        def _(j):
            rows0[i, pl.ds(j, 16)] = jnp.zeros((16,), jnp.float32)

    base = s * ROWS_PER_TILE
    nz = ROWS_PER_TILE // CHUNK
    rem = ROWS_PER_TILE - nz * CHUNK

    @pl.loop(0, nz)
    def _(q):
        pltpu.sync_copy(rows0, acc.at[pl.ds(base + q * CHUNK, CHUNK)])
    if rem:
        pltpu.sync_copy(rows0.at[pl.ds(0, rem)],
                        acc.at[pl.ds(base + nz * CHUNK, rem)])

    for cp in cps:
        cp.wait()
    plsc.subcore_barrier()

    def gather(ib_sel, row, slot):
        pltpu.async_copy(table.at[sib[ib_sel].at[row]], rows[slot],
                         gsem[slot])

    def step(o, p, c, first):
        b = (p * SBC + c) % 2
        pltpu.make_async_copy(table.at[sib[p].at[c]], rows[b],
                              gsem[b]).wait()
        pltpu.sync_copy(rows[b], acc.at[dib[p].at[c]], add=True)
        if c == 2:
            if p == 0:
                if not first:
                    pltpu.async_copy(src_hbm.at[wid, 2 * o + 1], sib1, is1)
                    pltpu.async_copy(dst_hbm.at[wid, 2 * o + 1], dib1, is1)
            else:
                pltpu.async_copy(src_hbm.at[wid, 2 * o + 2], sib0, is0)
                pltpu.async_copy(dst_hbm.at[wid, 2 * o + 2], dib0, is0)
        if c == 6:
            if p == 0:
                if not first:
                    pltpu.make_async_copy(src_hbm.at[wid, 2 * o + 1], sib1,
                                          is1).wait()
                    pltpu.make_async_copy(dst_hbm.at[wid, 2 * o + 1], dib1,
                                          is1).wait()
            else:
                pltpu.make_async_copy(src_hbm.at[wid, 2 * o + 2], sib0,
                                      is0).wait()
                pltpu.make_async_copy(dst_hbm.at[wid, 2 * o + 2], dib0,
                                      is0).wait()
        if c < 6:
            gather(p, c + 2, b)
        else:
            gather(1 - p, c - 6, b)

    gather(0, 0, 0)
    gather(0, 1, 1)

    for p in range(2):
        for cc in range(SBC):
            step(0, p, cc, first=True)

    @pl.loop(1, NO)
    def _(o):
        for p in range(2):
            for cc in range(SBC):
                step(o, p, cc, first=False)

    pltpu.make_async_copy(table.at[sib0.at[0]], rows0, gs0).wait()
    pltpu.make_async_copy(table.at[sib0.at[1]], rows1, gs1).wait()

    plsc.subcore_barrier()

    pltpu.sync_copy(acc.at[pl.ds(base, ROWS_PER_TILE)],
                    out.at[c, pl.ds(base, ROWS_PER_TILE)])


@jax.jit
def _sc_agg(table, src_hbm, dst_hbm):
    mesh = plsc.VectorSubcoreMesh(core_axis_name="c", subcore_axis_name="s")
    kfn = pl.kernel(
        _sc_agg_body,
        out_type=jax.ShapeDtypeStruct((NC, N_ACC, D), jnp.float32),
        mesh=mesh,
        scratch_types=(
            [pltpu.VMEM((CHUNK, D), jnp.float32) for _ in range(2)]
            + [pltpu.VMEM((SBC, CHUNK), jnp.int32) for _ in range(4)]
            + [pltpu.VMEM_SHARED((N_ACC, D), jnp.float32)]
            + [pltpu.SemaphoreType.DMA for _ in range(4)]
        ),
        compiler_params=pltpu.CompilerParams(use_tc_tiling_on_sc=True),
    )
    return kfn(table, src_hbm, dst_hbm)


def _sc_cnt_body(dst_hbm, out, dst_v, ones_v, zero_v, acc, isem):
    c = lax.axis_index("c")
    s = lax.axis_index("s")
    wid = s * NC + c

    cp = pltpu.async_copy(dst_hbm.at[wid], dst_v, isem)

    @pl.loop(0, CHUNK)
    def _(i):
        ones_v[i, pl.ds(0, CNT_W)] = jnp.ones((CNT_W,), jnp.float32)
        zero_v[i, pl.ds(0, CNT_W)] = jnp.zeros((CNT_W,), jnp.float32)

    base = s * ROWS_PER_TILE
    nz = ROWS_PER_TILE // CHUNK
    rem = ROWS_PER_TILE - nz * CHUNK

    @pl.loop(0, nz)
    def _(q):
        pltpu.sync_copy(zero_v, acc.at[pl.ds(base + q * CHUNK, CHUNK)])
    if rem:
        pltpu.sync_copy(zero_v.at[pl.ds(0, rem)],
                        acc.at[pl.ds(base + nz * CHUNK, rem)])

    cp.wait()
    plsc.subcore_barrier()

    @pl.loop(0, K)
    def _(j):
        pltpu.sync_copy(ones_v, acc.at[dst_v.at[j]], add=True)

    plsc.subcore_barrier()
    pltpu.sync_copy(acc.at[pl.ds(base, ROWS_PER_TILE)],
                    out.at[c, pl.ds(base, ROWS_PER_TILE)])


@jax.jit
def _sc_cnt(dst_flat):
    mesh = plsc.VectorSubcoreMesh(core_axis_name="c", subcore_axis_name="s")
    kfn = pl.kernel(
        _sc_cnt_body,
        out_type=jax.ShapeDtypeStruct((NC, N_ACC, CNT_W), jnp.float32),
        mesh=mesh,
        scratch_types=[
            pltpu.VMEM((K, CHUNK), jnp.int32),
            pltpu.VMEM((CHUNK, CNT_W), jnp.float32),
            pltpu.VMEM((CHUNK, CNT_W), jnp.float32),
            pltpu.VMEM_SHARED((N_ACC, CNT_W), jnp.float32),
            pltpu.SemaphoreType.DMA,
        ],
        compiler_params=pltpu.CompilerParams(use_tc_tiling_on_sc=False),
    )
    return kfn(dst_flat)


def _tc_pre_body(x_ref, wl_ref, wr_ref, b_ref, y_ref, r_ref):
    xv = x_ref[...]
    y_ref[...] = jnp.dot(xv, wl_ref[...], preferred_element_type=jnp.float32)
    r_ref[...] = jnp.dot(xv, wr_ref[...], preferred_element_type=jnp.float32) + b_ref[...]


@jax.jit
def _tc_pre(x, wl_t, wr_t, b):
    return pl.pallas_call(
        _tc_pre_body,
        out_shape=(jax.ShapeDtypeStruct((N, D), jnp.float32),
                   jax.ShapeDtypeStruct((N, D), jnp.float32)),
    )(x, wl_t, wr_t, b)


def _mean_from_partials(p_ref, c_ref):
    ssum = p_ref[0, :N, :] + p_ref[1, :N, :]
    cnt16 = c_ref[0, :N, :] + c_ref[1, :N, :]
    cnt = jnp.max(cnt16, axis=1, keepdims=True)
    return ssum / jnp.maximum(cnt, 1.0)


def _tc_mid_body(p_ref, c_ref, r1_ref, wl_ref, wr_ref, b_ref, y_ref, r2_ref):
    h = jnp.maximum(_mean_from_partials(p_ref, c_ref) + r1_ref[...], 0.0)
    y_ref[...] = jnp.dot(h, wl_ref[...], preferred_element_type=jnp.float32)
    r2_ref[...] = jnp.dot(h, wr_ref[...], preferred_element_type=jnp.float32) + b_ref[...]


@jax.jit
def _tc_mid(p1, cnt, r1, wl_t, wr_t, b):
    return pl.pallas_call(
        _tc_mid_body,
        out_shape=(jax.ShapeDtypeStruct((N, D), jnp.float32),
                   jax.ShapeDtypeStruct((N, D), jnp.float32)),
    )(p1, cnt, r1, wl_t, wr_t, b)


def _tc_post_body(p_ref, c_ref, r2_ref, out_ref):
    out_ref[...] = _mean_from_partials(p_ref, c_ref) + r2_ref[...]


@jax.jit
def _tc_post(p2, cnt, r2):
    return pl.pallas_call(
        _tc_post_body,
        out_shape=jax.ShapeDtypeStruct((N, D), jnp.float32),
    )(p2, cnt, r2)


def kernel(x, edgeIndex, W1_l, b1_l, W1_r, W2_l, b2_l, W2_r):
    src = edgeIndex[0]
    dst = edgeIndex[1]
    pad = E_PAD - E
    ntail = NW * SBC * CHUNK
    pad_src = (jnp.arange(pad, dtype=src.dtype) * 37) % N
    tail_src = (jnp.arange(ntail, dtype=src.dtype) * 53) % N
    srcp = jnp.concatenate([src, pad_src]).reshape(NW, K, CHUNK)
    srcp = jnp.concatenate([srcp, tail_src.reshape(NW, SBC, CHUNK)], axis=1)
    srcp = srcp.reshape(NW, NSB, SBC, CHUNK)
    pad_dst = N + (jnp.arange(pad, dtype=dst.dtype) % (N_ACC - N))
    tail_dst = N + (jnp.arange(ntail, dtype=dst.dtype) % (N_ACC - N))
    dstp = jnp.concatenate([dst, pad_dst]).reshape(NW, K, CHUNK)
    dstp_sb = jnp.concatenate([dstp, tail_dst.reshape(NW, SBC, CHUNK)],
                              axis=1).reshape(NW, NSB, SBC, CHUNK)

    cnt = _sc_cnt(dstp)
    y1, r1 = _tc_pre(x, W1_l.T, W1_r.T, b1_l[None, :])
    p1 = _sc_agg(y1, srcp, dstp_sb)
    y2, r2 = _tc_mid(p1, cnt, r1, W2_l.T, W2_r.T, b2_l[None, :])
    p2 = _sc_agg(y2, srcp, dstp_sb)
    return _tc_post(p2, cnt, r2)


# speedup vs baseline: 1.4580x; 1.0265x over previous
import jax
import jax.numpy as jnp
from jax import lax
from jax.experimental import pallas as pl
from jax.experimental.pallas import tpu as pltpu
from jax.experimental.pallas import tpu_sc as plsc

N = 10000
D = 128
E = 320000

NC = 2
NS = 16
NW = NC * NS
CHUNK = 128
SBC = 8
K = 80
SB = K // SBC
NSB = SB + 1
NO = SB // 2
E_PAD = NW * CHUNK * K
N_ACC = 10112
ROWS_PER_TILE = N_ACC // NS
CNT_W = 16


def _sc_agg_body(table, idx_hbm, out, rows0, rows1, sib0, sib1,
                 dib0, dib1, acc, is0, is1, gs0, gs1):
    c = lax.axis_index("c")
    s = lax.axis_index("s")
    wid = s * NC + c
    rows = [rows0, rows1]
    sib = [sib0, sib1]
    dib = [dib0, dib1]
    isem = [is0, is1]
    gsem = [gs0, gs1]

    cbase = wid * K

    def ld_sb(kind, sb_idx, ib_ref, sem):
        return pltpu.async_copy(
            idx_hbm.at[kind, pl.ds(cbase + sb_idx * SBC, SBC)], ib_ref, sem)

    def ld_sb_wait(kind, sb_idx, ib_ref, sem):
        pltpu.make_async_copy(
            idx_hbm.at[kind, pl.ds(cbase + sb_idx * SBC, SBC)], ib_ref,
            sem).wait()

    cps = [ld_sb(0, 0, sib0, is0), ld_sb(1, 0, dib0, is0),
           ld_sb(0, 1, sib1, is1), ld_sb(1, 1, dib1, is1)]

    @pl.loop(0, CHUNK)
    def _(i):
        @pl.loop(0, D, step=16)
        def _(j):
            rows0[i, pl.ds(j, 16)] = jnp.zeros((16,), jnp.float32)

    base = s * ROWS_PER_TILE
    nz = ROWS_PER_TILE // CHUNK
    rem = ROWS_PER_TILE - nz * CHUNK

    @pl.loop(0, nz)
    def _(q):
        pltpu.sync_copy(rows0, acc.at[pl.ds(base + q * CHUNK, CHUNK)])
    if rem:
        pltpu.sync_copy(rows0.at[pl.ds(0, rem)],
                        acc.at[pl.ds(base + nz * CHUNK, rem)])

    for cp in cps:
        cp.wait()
    plsc.subcore_barrier()

    def gather(ib_sel, row, slot):
        pltpu.async_copy(table.at[sib[ib_sel].at[row]], rows[slot],
                         gsem[slot])

    def step(o, p, c, first=False, last=False):
        b = (p * SBC + c) % 2
        pltpu.make_async_copy(table.at[sib[p].at[c]], rows[b],
                              gsem[b]).wait()
        pltpu.sync_copy(rows[b], acc.at[dib[p].at[c]], add=True)
        if c == 2:
            if p == 0:
                if not first:
                    ld_sb(0, 2 * o + 1, sib1, is1)
                    ld_sb(1, 2 * o + 1, dib1, is1)
            elif not last:
                ld_sb(0, 2 * o + 2, sib0, is0)
                ld_sb(1, 2 * o + 2, dib0, is0)
        if c == 6:
            if p == 0:
                if not first:
                    ld_sb_wait(0, 2 * o + 1, sib1, is1)
                    ld_sb_wait(1, 2 * o + 1, dib1, is1)
            elif not last:
                ld_sb_wait(0, 2 * o + 2, sib0, is0)
                ld_sb_wait(1, 2 * o + 2, dib0, is0)
        if c < 6:
            gather(p, c + 2, b)
        elif not (last and p == 1):
            gather(1 - p, c - 6, b)

    gather(0, 0, 0)
    gather(0, 1, 1)

    for p in range(2):
        for cc in range(SBC):
            step(0, p, cc, first=True)

    @pl.loop(1, NO - 1)
    def _(o):
        for p in range(2):
            for cc in range(SBC):
                step(o, p, cc)

    for p in range(2):
        for cc in range(SBC):
            step(NO - 1, p, cc, last=True)

    plsc.subcore_barrier()

    pltpu.sync_copy(acc.at[pl.ds(base, ROWS_PER_TILE)],
                    out.at[c, pl.ds(base, ROWS_PER_TILE)])


@jax.jit
def _sc_agg(table, idx_hbm):
    mesh = plsc.VectorSubcoreMesh(core_axis_name="c", subcore_axis_name="s")
    kfn = pl.kernel(
        _sc_agg_body,
        out_type=jax.ShapeDtypeStruct((NC, N_ACC, D), jnp.float32),
        mesh=mesh,
        scratch_types=(
            [pltpu.VMEM((CHUNK, D), jnp.float32) for _ in range(2)]
            + [pltpu.VMEM((SBC, CHUNK), jnp.int32) for _ in range(4)]
            + [pltpu.VMEM_SHARED((N_ACC, D), jnp.float32)]
            + [pltpu.SemaphoreType.DMA for _ in range(4)]
        ),
        compiler_params=pltpu.CompilerParams(use_tc_tiling_on_sc=True),
    )
    return kfn(table, idx_hbm)


def _sc_cnt_body(idx_hbm, out, dst_v, ones_v, zero_v, acc, isem):
    c = lax.axis_index("c")
    s = lax.axis_index("s")
    wid = s * NC + c

    cp = pltpu.async_copy(idx_hbm.at[1, pl.ds(wid * K, K)], dst_v, isem)

    @pl.loop(0, CHUNK)
    def _(i):
        ones_v[i, pl.ds(0, CNT_W)] = jnp.ones((CNT_W,), jnp.float32)
        zero_v[i, pl.ds(0, CNT_W)] = jnp.zeros((CNT_W,), jnp.float32)

    base = s * ROWS_PER_TILE
    nz = ROWS_PER_TILE // CHUNK
    rem = ROWS_PER_TILE - nz * CHUNK

    @pl.loop(0, nz)
    def _(q):
        pltpu.sync_copy(zero_v, acc.at[pl.ds(base + q * CHUNK, CHUNK)])
    if rem:
        pltpu.sync_copy(zero_v.at[pl.ds(0, rem)],
                        acc.at[pl.ds(base + nz * CHUNK, rem)])

    cp.wait()
    plsc.subcore_barrier()

    @pl.loop(0, K)
    def _(j):
        pltpu.sync_copy(ones_v, acc.at[dst_v.at[j]], add=True)

    plsc.subcore_barrier()
    pltpu.sync_copy(acc.at[pl.ds(base, ROWS_PER_TILE)],
                    out.at[c, pl.ds(base, ROWS_PER_TILE)])


@jax.jit
def _sc_cnt(idx_hbm):
    mesh = plsc.VectorSubcoreMesh(core_axis_name="c", subcore_axis_name="s")
    kfn = pl.kernel(
        _sc_cnt_body,
        out_type=jax.ShapeDtypeStruct((NC, N_ACC, CNT_W), jnp.float32),
        mesh=mesh,
        scratch_types=[
            pltpu.VMEM((K, CHUNK), jnp.int32),
            pltpu.VMEM((CHUNK, CNT_W), jnp.float32),
            pltpu.VMEM((CHUNK, CNT_W), jnp.float32),
            pltpu.VMEM_SHARED((N_ACC, CNT_W), jnp.float32),
            pltpu.SemaphoreType.DMA,
        ],
        compiler_params=pltpu.CompilerParams(use_tc_tiling_on_sc=False),
    )
    return kfn(idx_hbm)


def _tc_pre_body(x_ref, wl_ref, wr_ref, b_ref, y_ref, r_ref):
    xv = x_ref[...]
    y_ref[...] = jnp.dot(xv, wl_ref[...], preferred_element_type=jnp.float32)
    r_ref[...] = jnp.dot(xv, wr_ref[...], preferred_element_type=jnp.float32) + b_ref[...]


@jax.jit
def _tc_pre(x, wl_t, wr_t, b):
    return pl.pallas_call(
        _tc_pre_body,
        out_shape=(jax.ShapeDtypeStruct((N, D), jnp.float32),
                   jax.ShapeDtypeStruct((N, D), jnp.float32)),
    )(x, wl_t, wr_t, b)


def _mean_from_partials(p_ref, c_ref):
    ssum = p_ref[0, :N, :] + p_ref[1, :N, :]
    cnt16 = c_ref[0, :N, :] + c_ref[1, :N, :]
    cnt = jnp.max(cnt16, axis=1, keepdims=True)
    return ssum / jnp.maximum(cnt, 1.0)


def _tc_mid_body(p_ref, c_ref, r1_ref, wl_ref, wr_ref, b_ref, y_ref, r2_ref):
    h = jnp.maximum(_mean_from_partials(p_ref, c_ref) + r1_ref[...], 0.0)
    y_ref[...] = jnp.dot(h, wl_ref[...], preferred_element_type=jnp.float32)
    r2_ref[...] = jnp.dot(h, wr_ref[...], preferred_element_type=jnp.float32) + b_ref[...]


@jax.jit
def _tc_mid(p1, cnt, r1, wl_t, wr_t, b):
    return pl.pallas_call(
        _tc_mid_body,
        out_shape=(jax.ShapeDtypeStruct((N, D), jnp.float32),
                   jax.ShapeDtypeStruct((N, D), jnp.float32)),
    )(p1, cnt, r1, wl_t, wr_t, b)


def _tc_post_body(p_ref, c_ref, r2_ref, out_ref):
    out_ref[...] = _mean_from_partials(p_ref, c_ref) + r2_ref[...]


@jax.jit
def _tc_post(p2, cnt, r2):
    return pl.pallas_call(
        _tc_post_body,
        out_shape=jax.ShapeDtypeStruct((N, D), jnp.float32),
    )(p2, cnt, r2)


def kernel(x, edgeIndex, W1_l, b1_l, W1_r, W2_l, b2_l, W2_r):
    pad = E_PAD - E
    pad_src = (jnp.arange(pad, dtype=jnp.int32) * 37) % N
    pad_dst = N + (jnp.arange(pad, dtype=jnp.int32) % (N_ACC - N))
    idx = jnp.concatenate([edgeIndex, jnp.stack([pad_src, pad_dst])], axis=1)
    idx = idx.reshape(2, NW * K, CHUNK)

    cnt = _sc_cnt(idx)
    y1, r1 = _tc_pre(x, W1_l.T, W1_r.T, b1_l[None, :])
    p1 = _sc_agg(y1, idx)
    y2, r2 = _tc_mid(p1, cnt, r1, W2_l.T, W2_r.T, b2_l[None, :])
    p2 = _sc_agg(y2, idx)
    return _tc_post(p2, cnt, r2)
